# baseline (speedup 1.0000x reference)
import functools

import jax
import jax.numpy as jnp
from jax import lax
from jax.experimental import pallas as pl
from jax.experimental.pallas import tpu as pltpu
from jax.experimental.pallas import tpu_sc as plsc

_N = 10000
_E = 320000
_NF = 128
_NORM = 100.0

_NC = 2
_NS = 16
_NW = _NC * _NS
_EPW = _E // _NW
_K = 80
_NCH = _EPW // _K
_RPT = 624


def _sc_gather_combine(ta, tb, row, col, D, sign, dtype, tct=True):
    bf16 = dtype == jnp.bfloat16
    W = D // 2 if bf16 else D
    G = D // 32 if bf16 else D // 16
    sdt = jnp.int32 if bf16 else jnp.float32
    mesh = plsc.VectorSubcoreMesh(core_axis_name="c", subcore_axis_name="s")

    @functools.partial(
        pl.kernel,
        out_type=jax.ShapeDtypeStruct((_E, W), sdt),
        mesh=mesh,
        scratch_types=[
            pltpu.VMEM((_K,), jnp.int32), pltpu.VMEM((_K,), jnp.int32),
            pltpu.VMEM((_K,), jnp.int32), pltpu.VMEM((_K,), jnp.int32),
            pltpu.VMEM((_K, W), sdt), pltpu.VMEM((_K, W), sdt),
            pltpu.VMEM((_K, W), sdt), pltpu.VMEM((_K, W), sdt),
            pltpu.SemaphoreType.DMA, pltpu.SemaphoreType.DMA,
        ],
        compiler_params=pltpu.CompilerParams(
            needs_layout_passes=not bf16, use_tc_tiling_on_sc=tct),
    )
    def kern(ta_h, tb_h, row_h, col_h, out_h,
             ir0, ir1, ic0, ic1, a0, a1, b0, b1, s0, s1):
        wid = lax.axis_index("c") * _NS + lax.axis_index("s")
        base = wid * _EPW
        irs = (ir0, ir1)
        ics = (ic0, ic1)
        abufs = (a0, a1)
        bbufs = (b0, b1)
        sems = (s0, s1)

        def fire(ci, t):
            off = base + ci * _K
            pltpu.sync_copy(row_h.at[pl.ds(off, _K)], irs[t])
            pltpu.sync_copy(col_h.at[pl.ds(off, _K)], ics[t])
            pltpu.async_copy(ta_h.at[irs[t]], abufs[t], sems[t])
            pltpu.async_copy(tb_h.at[ics[t]], bbufs[t], sems[t])

        def waitproc(ci, t):
            pltpu.make_async_copy(ta_h.at[irs[t]], abufs[t], sems[t]).wait()
            pltpu.make_async_copy(tb_h.at[ics[t]], bbufs[t], sems[t]).wait()
            a = abufs[t]
            b = bbufs[t]

            def addrow(r, carry):
                for g in range(G):
                    av = a[r, pl.ds(g * 16, 16)]
                    bv = b[r, pl.ds(g * 16, 16)]
                    if bf16:
                        av = plsc.bitcast(av, jnp.bfloat16)
                        bv = plsc.bitcast(bv, jnp.bfloat16)
                    res = av + bv if sign > 0 else av - bv
                    if bf16:
                        res = plsc.bitcast(res, jnp.int32)
                    a[r, pl.ds(g * 16, 16)] = res
                return carry

            lax.fori_loop(0, _K, addrow, 0)
            off = base + ci * _K
            pltpu.sync_copy(a, out_h.at[pl.ds(off, _K)])

        fire(0, 0)

        def body(i, carry):
            j = i * 2
            fire(j + 1, 1)
            waitproc(j, 0)
            fire(j + 2, 0)
            waitproc(j + 1, 1)
            return carry

        lax.fori_loop(0, (_NCH - 1) // 2, body, 0)
        waitproc(_NCH - 1, 0)

    return kern(ta, tb, row, col)


def _sc_scatter_add(feat, row, zeros_tile, D, dtype, tct=True):
    mesh = plsc.VectorSubcoreMesh(core_axis_name="c", subcore_axis_name="s")

    @functools.partial(
        pl.kernel,
        out_type=jax.ShapeDtypeStruct((_NC, _N, D), dtype),
        mesh=mesh,
        scratch_types=[
            pltpu.VMEM((_K,), jnp.int32), pltpu.VMEM((_K,), jnp.int32),
            pltpu.VMEM((_K, D), dtype), pltpu.VMEM((_K, D), dtype),
            pltpu.VMEM_SHARED((_N, D), dtype),
            pltpu.SemaphoreType.DMA, pltpu.SemaphoreType.DMA,
        ],
        compiler_params=pltpu.CompilerParams(use_tc_tiling_on_sc=tct),
    )
    def kern(feat_h, row_h, z_h, out_h, i0, i1, f0, f1, acc, s0, s1):
        c = lax.axis_index("c")
        s = lax.axis_index("s")
        base = (c * _NS + s) * _EPW
        idxs = (i0, i1)
        fbufs = (f0, f1)
        sems = (s0, s1)

        pltpu.sync_copy(z_h.at[pl.ds(0, _RPT)], acc.at[pl.ds(s * _RPT, _RPT)])

        @pl.when(s == _NS - 1)
        def _():
            pltpu.sync_copy(z_h.at[pl.ds(_RPT, 16)],
                            acc.at[pl.ds(_NS * _RPT, 16)])

        plsc.subcore_barrier()

        def fire(ci, t):
            off = base + ci * _K
            pltpu.sync_copy(row_h.at[pl.ds(off, _K)], idxs[t])
            pltpu.async_copy(feat_h.at[pl.ds(off, _K)], fbufs[t], sems[t])

        def proc(ci, t):
            pltpu.make_async_copy(feat_h.at[pl.ds(0, _K)], fbufs[t], sems[t]).wait()
            pltpu.sync_copy(fbufs[t], acc.at[idxs[t]], add=True)

        fire(0, 0)

        def body(i, carry):
            j = i * 2
            fire(j + 1, 1)
            proc(j, 0)
            fire(j + 2, 0)
            proc(j + 1, 1)
            return carry

        lax.fori_loop(0, (_NCH - 1) // 2, body, 0)
        proc(_NCH - 1, 0)
        plsc.subcore_barrier()

        pltpu.sync_copy(acc.at[pl.ds(s * _RPT, _RPT)],
                        out_h.at[c, pl.ds(s * _RPT, _RPT)])

        @pl.when(s == _NS - 1)
        def _():
            pltpu.sync_copy(acc.at[pl.ds(_NS * _RPT, 16)],
                            out_h.at[c, pl.ds(_NS * _RPT, 16)])

    return kern(feat, row, zeros_tile)


_BN = 2000
_BE = 2000


def _node_pre_body(h_ref, at_ref, bt_ref, ba_ref, ha_ref, hb_ref):
    h = h_ref[...]
    ha = jnp.dot(h, at_ref[...], preferred_element_type=jnp.float32) + ba_ref[...]
    hb = jnp.dot(h, bt_ref[...], preferred_element_type=jnp.float32)
    ha_ref[...] = ha.astype(jnp.bfloat16)
    hb_ref[...] = hb.astype(jnp.bfloat16)


def _tc_node_pre(h, at, bt, ba):
    return pl.pallas_call(
        _node_pre_body,
        grid=(_N // _BN,),
        in_specs=[
            pl.BlockSpec((_BN, _NF), lambda i: (i, 0)),
            pl.BlockSpec((_NF, _NF), lambda i: (0, 0)),
            pl.BlockSpec((_NF, _NF), lambda i: (0, 0)),
            pl.BlockSpec((1, _NF), lambda i: (0, 0)),
        ],
        out_specs=[
            pl.BlockSpec((_BN, _NF), lambda i: (i, 0)),
            pl.BlockSpec((_BN, _NF), lambda i: (i, 0)),
        ],
        out_shape=[jax.ShapeDtypeStruct((_N, _NF), jnp.bfloat16)] * 2,
    )(h, at, bt, ba)


def _geom_body(xd_ref, eat_ref, geo_ref):
    xd = xd_ref[...]
    r2 = jnp.sum(xd * xd, axis=1, keepdims=True)
    cd = xd / (jnp.sqrt(r2 + 1e-8) + 1.0)
    z = jnp.zeros((xd.shape[0], 11), jnp.float32)
    geo_ref[...] = jnp.concatenate([r2, eat_ref[...], cd[:, 0:3], z], axis=1)


def _tc_geom(xd, eattr):
    return pl.pallas_call(
        _geom_body,
        grid=(_E // _BE,),
        in_specs=[
            pl.BlockSpec((_BE, 16), lambda i: (i, 0)),
            pl.BlockSpec((_BE, 1), lambda i: (i, 0)),
        ],
        out_specs=pl.BlockSpec((_BE, 16), lambda i: (i, 0)),
        out_shape=jax.ShapeDtypeStruct((_E, 16), jnp.float32),
    )(xd, eattr)


def _edge_mlp_body(e0_ref, geo_ref, ct_ref, w1t_ref, b1_ref, aw_ref, ab_ref, out_ref):
    e0 = e0_ref[...].astype(jnp.float32)
    ea = geo_ref[...][:, 0:2]
    t0 = e0 + jnp.dot(ea, ct_ref[...], preferred_element_type=jnp.float32)
    t0 = t0 * jax.nn.sigmoid(t0)
    t1 = jnp.dot(t0, w1t_ref[...], preferred_element_type=jnp.float32) + b1_ref[...]
    t1 = t1 * jax.nn.sigmoid(t1)
    av = jnp.dot(t1, aw_ref[...], preferred_element_type=jnp.float32) + ab_ref[...]
    out_ref[...] = t1 * jax.nn.sigmoid(av)


def _tc_edge_mlp(e0, geo, ct, w1t, b1, aw, ab):
    return pl.pallas_call(
        _edge_mlp_body,
        grid=(_E // _BE,),
        in_specs=[
            pl.BlockSpec((_BE, _NF), lambda i: (i, 0)),
            pl.BlockSpec((_BE, 16), lambda i: (i, 0)),
            pl.BlockSpec((2, _NF), lambda i: (0, 0)),
            pl.BlockSpec((_NF, _NF), lambda i: (0, 0)),
            pl.BlockSpec((1, _NF), lambda i: (0, 0)),
            pl.BlockSpec((_NF, 1), lambda i: (0, 0)),
            pl.BlockSpec((1, 1), lambda i: (0, 0)),
        ],
        out_specs=pl.BlockSpec((_BE, _NF), lambda i: (i, 0)),
        out_shape=jax.ShapeDtypeStruct((_E, _NF), jnp.float32),
    )(e0, geo, ct, w1t, b1, aw, ab)


def _node_mlp_body(h_ref, p0_ref, p1_ref, ut_ref, vt_ref, b0_ref, w1t_ref, b1_ref, out_ref):
    h = h_ref[...]
    agg = (p0_ref[...].astype(jnp.float32)
           + p1_ref[...].astype(jnp.float32)) * (1.0 / _NORM)
    t = (jnp.dot(h, ut_ref[...], preferred_element_type=jnp.float32)
         + jnp.dot(agg, vt_ref[...], preferred_element_type=jnp.float32)
         + b0_ref[...])
    t = t * jax.nn.sigmoid(t)
    dh = jnp.dot(t, w1t_ref[...], preferred_element_type=jnp.float32) + b1_ref[...]
    out_ref[...] = h + dh


def _tc_node_mlp(h, p0, p1, ut, vt, b0, w1t, b1):
    return pl.pallas_call(
        _node_mlp_body,
        grid=(_N // _BN,),
        in_specs=[
            pl.BlockSpec((_BN, _NF), lambda i: (i, 0)),
            pl.BlockSpec((_BN, _NF), lambda i: (i, 0)),
            pl.BlockSpec((_BN, _NF), lambda i: (i, 0)),
            pl.BlockSpec((_NF, _NF), lambda i: (0, 0)),
            pl.BlockSpec((_NF, _NF), lambda i: (0, 0)),
            pl.BlockSpec((1, _NF), lambda i: (0, 0)),
            pl.BlockSpec((_NF, _NF), lambda i: (0, 0)),
            pl.BlockSpec((1, _NF), lambda i: (0, 0)),
        ],
        out_specs=pl.BlockSpec((_BN, _NF), lambda i: (i, 0)),
        out_shape=jax.ShapeDtypeStruct((_N, _NF), jnp.float32),
    )(h, p0, p1, ut, vt, b0, w1t, b1)


def _coord_edge_body(c0_ref, geo_ref, ct_ref, w1t_ref, b1_ref, w2t_ref, out_ref):
    geo = geo_ref[...]
    ea = geo[:, 0:2]
    t0 = c0_ref[...].astype(jnp.float32) + jnp.dot(ea, ct_ref[...], preferred_element_type=jnp.float32)
    t0 = t0 * jax.nn.sigmoid(t0)
    t1 = jnp.dot(t0, w1t_ref[...], preferred_element_type=jnp.float32) + b1_ref[...]
    t1 = t1 * jax.nn.sigmoid(t1)
    tt = jnp.dot(t1, w2t_ref[...], preferred_element_type=jnp.float32)
    z = jnp.zeros((geo.shape[0], 13), jnp.float32)
    out_ref[...] = jnp.concatenate([geo[:, 2:5] * tt, z], axis=1)


def _tc_coord_edge(c0, geo, ct, w1t, b1, w2t):
    return pl.pallas_call(
        _coord_edge_body,
        grid=(_E // _BE,),
        in_specs=[
            pl.BlockSpec((_BE, _NF), lambda i: (i, 0)),
            pl.BlockSpec((_BE, 16), lambda i: (i, 0)),
            pl.BlockSpec((2, _NF), lambda i: (0, 0)),
            pl.BlockSpec((_NF, _NF), lambda i: (0, 0)),
            pl.BlockSpec((1, _NF), lambda i: (0, 0)),
            pl.BlockSpec((_NF, 1), lambda i: (0, 0)),
        ],
        out_specs=pl.BlockSpec((_BE, 16), lambda i: (i, 0)),
        out_shape=jax.ShapeDtypeStruct((_E, 16), jnp.float32),
    )(c0, geo, ct, w1t, b1, w2t)


def _coord_apply_body(x_ref, q0_ref, q1_ref, out_ref):
    q = (q0_ref[...].astype(jnp.float32)
         + q1_ref[...].astype(jnp.float32)) * (1.0 / _NORM)
    out_ref[...] = x_ref[...] + q[:, 0:3]


def _tc_coord_apply(x, q0, q1):
    return pl.pallas_call(
        _coord_apply_body,
        grid=(_N // _BN,),
        in_specs=[
            pl.BlockSpec((_BN, 3), lambda i: (i, 0)),
            pl.BlockSpec((_BN, 16), lambda i: (i, 0)),
            pl.BlockSpec((_BN, 16), lambda i: (i, 0)),
        ],
        out_specs=pl.BlockSpec((_BN, 3), lambda i: (i, 0)),
        out_shape=jax.ShapeDtypeStruct((_N, 3), jnp.float32),
    )(x, q0, q1)


def _pack_bf16(a):
    return jax.lax.bitcast_convert_type(
        a.reshape(a.shape[0], a.shape[1] // 2, 2), jnp.int32)


def _unpack_bf16(a):
    return jax.lax.bitcast_convert_type(a, jnp.bfloat16).reshape(a.shape[0], -1)


def kernel(h, x, edge_index, edge_attr, params):
    row = edge_index[0]
    col = edge_index[1]
    x16 = jnp.concatenate([x, jnp.zeros((_N, 13), jnp.float32)], axis=1)
    zeros128 = jnp.zeros((_RPT + 16, _NF), jnp.float32)
    zeros16 = jnp.zeros((_RPT + 16, 16), jnp.float32)

    xd = _sc_gather_combine(x16, x16, row, col, 16, -1, jnp.float32,
                            tct=False)
    geo = _tc_geom(xd, edge_attr)

    for i in range(2):
        w0 = params[f"gcl{i}_e_W0"]
        at = w0[:, :_NF].T
        bt = w0[:, _NF:2 * _NF].T
        ct = w0[:, 2 * _NF:].T
        ha, hb = _tc_node_pre(h, at, bt, params[f"gcl{i}_e_b0"][None, :])
        e0 = _unpack_bf16(_sc_gather_combine(_pack_bf16(ha), _pack_bf16(hb),
                                             row, col, _NF, 1, jnp.bfloat16,
                                             tct=False))
        ef = _tc_edge_mlp(
            e0, geo, ct,
            params[f"gcl{i}_e_W1"].T,
            params[f"gcl{i}_e_b1"][None, :],
            params[f"gcl{i}_att_W"].T,
            params[f"gcl{i}_att_b"][None, :],
        )
        parts = _sc_scatter_add(ef, row, zeros128, _NF, jnp.float32)
        nw0 = params[f"gcl{i}_n_W0"]
        h = _tc_node_mlp(
            h, parts[0], parts[1],
            nw0[:, :_NF].T, nw0[:, _NF:].T,
            params[f"gcl{i}_n_b0"][None, :],
            params[f"gcl{i}_n_W1"].T,
            params[f"gcl{i}_n_b1"][None, :],
        )

    cw0 = params["c_W0"]
    ca, cb = _tc_node_pre(h, cw0[:, :_NF].T, cw0[:, _NF:2 * _NF].T,
                          params["c_b0"][None, :])
    c0 = _unpack_bf16(_sc_gather_combine(_pack_bf16(ca), _pack_bf16(cb),
                                         row, col, _NF, 1, jnp.bfloat16,
                                         tct=False))
    trans = _tc_coord_edge(
        c0, geo, cw0[:, 2 * _NF:].T,
        params["c_W1"].T,
        params["c_b1"][None, :],
        params["c_W2"].T,
    )
    qparts = _sc_scatter_add(trans, row, zeros16, 16, jnp.float32,
                             tct=False)
    x_new = _tc_coord_apply(x, qparts[0], qparts[1])
    return h, x_new


# speedup vs baseline: 1.9969x; 1.9969x over previous
import functools

import jax
import jax.numpy as jnp
from jax import lax
from jax.experimental import pallas as pl
from jax.experimental.pallas import tpu as pltpu
from jax.experimental.pallas import tpu_sc as plsc

_N = 10000
_E = 320000
_NF = 128
_NORM = 100.0

_NC = 2
_NS = 16
_NW = _NC * _NS
_EPW = _E // _NW
_K = 80
_NCH = _EPW // _K
_RPT = 624


def _sc_gather_combine(ta, tb, row, col, D, sign, dtype, tct=True):
    bf16 = dtype == jnp.bfloat16
    W = D // 2 if bf16 else D
    G = D // 32 if bf16 else D // 16
    sdt = jnp.int32 if bf16 else jnp.float32
    mesh = plsc.VectorSubcoreMesh(core_axis_name="c", subcore_axis_name="s")

    @functools.partial(
        pl.kernel,
        out_type=jax.ShapeDtypeStruct((_E, W), sdt),
        mesh=mesh,
        scratch_types=[
            pltpu.VMEM((_K,), jnp.int32), pltpu.VMEM((_K,), jnp.int32),
            pltpu.VMEM((_K,), jnp.int32), pltpu.VMEM((_K,), jnp.int32),
            pltpu.VMEM((_K, W), sdt), pltpu.VMEM((_K, W), sdt),
            pltpu.VMEM((_K, W), sdt), pltpu.VMEM((_K, W), sdt),
            pltpu.SemaphoreType.DMA, pltpu.SemaphoreType.DMA,
        ],
        compiler_params=pltpu.CompilerParams(
            needs_layout_passes=not bf16, use_tc_tiling_on_sc=tct),
    )
    def kern(ta_h, tb_h, row_h, col_h, out_h,
             ir0, ir1, ic0, ic1, a0, a1, b0, b1, s0, s1):
        wid = lax.axis_index("c") * _NS + lax.axis_index("s")
        base = wid * _EPW
        irs = (ir0, ir1)
        ics = (ic0, ic1)
        abufs = (a0, a1)
        bbufs = (b0, b1)
        sems = (s0, s1)

        def fire(ci, t):
            off = base + ci * _K
            pltpu.sync_copy(row_h.at[pl.ds(off, _K)], irs[t])
            pltpu.sync_copy(col_h.at[pl.ds(off, _K)], ics[t])
            pltpu.async_copy(ta_h.at[irs[t]], abufs[t], sems[t])
            pltpu.async_copy(tb_h.at[ics[t]], bbufs[t], sems[t])

        def waitproc(ci, t):
            pltpu.make_async_copy(ta_h.at[irs[t]], abufs[t], sems[t]).wait()
            pltpu.make_async_copy(tb_h.at[ics[t]], bbufs[t], sems[t]).wait()
            a = abufs[t]
            b = bbufs[t]

            def addrow(r, carry):
                for g in range(G):
                    av = a[r, pl.ds(g * 16, 16)]
                    bv = b[r, pl.ds(g * 16, 16)]
                    if bf16:
                        av = plsc.bitcast(av, jnp.bfloat16)
                        bv = plsc.bitcast(bv, jnp.bfloat16)
                    res = av + bv if sign > 0 else av - bv
                    if bf16:
                        res = plsc.bitcast(res, jnp.int32)
                    a[r, pl.ds(g * 16, 16)] = res
                return carry

            lax.fori_loop(0, _K, addrow, 0)
            off = base + ci * _K
            pltpu.sync_copy(a, out_h.at[pl.ds(off, _K)])

        fire(0, 0)

        def body(i, carry):
            j = i * 2
            fire(j + 1, 1)
            waitproc(j, 0)
            fire(j + 2, 0)
            waitproc(j + 1, 1)
            return carry

        lax.fori_loop(0, (_NCH - 1) // 2, body, 0)
        waitproc(_NCH - 1, 0)

    return kern(ta, tb, row, col)


def _sc_scatter_add(feat, row, zeros_tile, D, dtype, tct=True):
    mesh = plsc.VectorSubcoreMesh(core_axis_name="c", subcore_axis_name="s")

    @functools.partial(
        pl.kernel,
        out_type=jax.ShapeDtypeStruct((_NC, _N, D), dtype),
        mesh=mesh,
        scratch_types=[
            pltpu.VMEM((_K,), jnp.int32), pltpu.VMEM((_K,), jnp.int32),
            pltpu.VMEM((_K, D), dtype), pltpu.VMEM((_K, D), dtype),
            pltpu.VMEM_SHARED((_N, D), dtype),
            pltpu.SemaphoreType.DMA, pltpu.SemaphoreType.DMA,
        ],
        compiler_params=pltpu.CompilerParams(use_tc_tiling_on_sc=tct),
    )
    def kern(feat_h, row_h, z_h, out_h, i0, i1, f0, f1, acc, s0, s1):
        c = lax.axis_index("c")
        s = lax.axis_index("s")
        base = (c * _NS + s) * _EPW
        idxs = (i0, i1)
        fbufs = (f0, f1)
        sems = (s0, s1)

        pltpu.sync_copy(z_h.at[pl.ds(0, _RPT)], acc.at[pl.ds(s * _RPT, _RPT)])

        @pl.when(s == _NS - 1)
        def _():
            pltpu.sync_copy(z_h.at[pl.ds(_RPT, 16)],
                            acc.at[pl.ds(_NS * _RPT, 16)])

        plsc.subcore_barrier()

        def fire(ci, t):
            off = base + ci * _K
            pltpu.sync_copy(row_h.at[pl.ds(off, _K)], idxs[t])
            pltpu.async_copy(feat_h.at[pl.ds(off, _K)], fbufs[t], sems[t])

        def proc(ci, t):
            pltpu.make_async_copy(feat_h.at[pl.ds(0, _K)], fbufs[t], sems[t]).wait()
            pltpu.sync_copy(fbufs[t], acc.at[idxs[t]], add=True)

        fire(0, 0)

        def body(i, carry):
            j = i * 2
            fire(j + 1, 1)
            proc(j, 0)
            fire(j + 2, 0)
            proc(j + 1, 1)
            return carry

        lax.fori_loop(0, (_NCH - 1) // 2, body, 0)
        proc(_NCH - 1, 0)
        plsc.subcore_barrier()

        pltpu.sync_copy(acc.at[pl.ds(s * _RPT, _RPT)],
                        out_h.at[c, pl.ds(s * _RPT, _RPT)])

        @pl.when(s == _NS - 1)
        def _():
            pltpu.sync_copy(acc.at[pl.ds(_NS * _RPT, 16)],
                            out_h.at[c, pl.ds(_NS * _RPT, 16)])

    return kern(feat, row, zeros_tile)


_BN = 2000
_BE = 2000


def _node_pre_body(h_ref, at_ref, bt_ref, ba_ref, ha_ref, hb_ref):
    h = h_ref[...]
    ha_ref[...] = jnp.dot(h, at_ref[...], preferred_element_type=jnp.float32) + ba_ref[...]
    hb_ref[...] = jnp.dot(h, bt_ref[...], preferred_element_type=jnp.float32)


def _tc_node_pre(h, at, bt, ba):
    return pl.pallas_call(
        _node_pre_body,
        grid=(_N // _BN,),
        in_specs=[
            pl.BlockSpec((_BN, _NF), lambda i: (i, 0)),
            pl.BlockSpec((_NF, _NF), lambda i: (0, 0)),
            pl.BlockSpec((_NF, _NF), lambda i: (0, 0)),
            pl.BlockSpec((1, _NF), lambda i: (0, 0)),
        ],
        out_specs=[
            pl.BlockSpec((_BN, _NF), lambda i: (i, 0)),
            pl.BlockSpec((_BN, _NF), lambda i: (i, 0)),
        ],
        out_shape=[jax.ShapeDtypeStruct((_N, _NF), jnp.float32)] * 2,
    )(h, at, bt, ba)


def _geom_body(xd_ref, eat_ref, geo_ref):
    xd = xd_ref[...]
    r2 = jnp.sum(xd * xd, axis=1, keepdims=True)
    cd = xd / (jnp.sqrt(r2 + 1e-8) + 1.0)
    z = jnp.zeros((xd.shape[0], 11), jnp.float32)
    geo_ref[...] = jnp.concatenate([r2, eat_ref[...], cd[:, 0:3], z], axis=1)


def _tc_geom(xd, eattr):
    return pl.pallas_call(
        _geom_body,
        grid=(_E // _BE,),
        in_specs=[
            pl.BlockSpec((_BE, 16), lambda i: (i, 0)),
            pl.BlockSpec((_BE, 1), lambda i: (i, 0)),
        ],
        out_specs=pl.BlockSpec((_BE, 16), lambda i: (i, 0)),
        out_shape=jax.ShapeDtypeStruct((_E, 16), jnp.float32),
    )(xd, eattr)


def _edge_mlp_body(e0_ref, geo_ref, ct_ref, w1t_ref, b1_ref, aw_ref, ab_ref, out_ref):
    e0 = e0_ref[...].astype(jnp.float32)
    ea = geo_ref[...][:, 0:2]
    t0 = e0 + jnp.dot(ea, ct_ref[...], preferred_element_type=jnp.float32)
    t0 = t0 * jax.nn.sigmoid(t0)
    t1 = jnp.dot(t0, w1t_ref[...], preferred_element_type=jnp.float32) + b1_ref[...]
    t1 = t1 * jax.nn.sigmoid(t1)
    av = jnp.dot(t1, aw_ref[...], preferred_element_type=jnp.float32) + ab_ref[...]
    out_ref[...] = t1 * jax.nn.sigmoid(av)


def _tc_edge_mlp(e0, geo, ct, w1t, b1, aw, ab):
    return pl.pallas_call(
        _edge_mlp_body,
        grid=(_E // _BE,),
        in_specs=[
            pl.BlockSpec((_BE, _NF), lambda i: (i, 0)),
            pl.BlockSpec((_BE, 16), lambda i: (i, 0)),
            pl.BlockSpec((2, _NF), lambda i: (0, 0)),
            pl.BlockSpec((_NF, _NF), lambda i: (0, 0)),
            pl.BlockSpec((1, _NF), lambda i: (0, 0)),
            pl.BlockSpec((_NF, 1), lambda i: (0, 0)),
            pl.BlockSpec((1, 1), lambda i: (0, 0)),
        ],
        out_specs=pl.BlockSpec((_BE, _NF), lambda i: (i, 0)),
        out_shape=jax.ShapeDtypeStruct((_E, _NF), jnp.float32),
    )(e0, geo, ct, w1t, b1, aw, ab)


def _node_mlp_body(h_ref, p0_ref, p1_ref, ut_ref, vt_ref, b0_ref, w1t_ref, b1_ref, out_ref):
    h = h_ref[...]
    agg = (p0_ref[...].astype(jnp.float32)
           + p1_ref[...].astype(jnp.float32)) * (1.0 / _NORM)
    t = (jnp.dot(h, ut_ref[...], preferred_element_type=jnp.float32)
         + jnp.dot(agg, vt_ref[...], preferred_element_type=jnp.float32)
         + b0_ref[...])
    t = t * jax.nn.sigmoid(t)
    dh = jnp.dot(t, w1t_ref[...], preferred_element_type=jnp.float32) + b1_ref[...]
    out_ref[...] = h + dh


def _tc_node_mlp(h, p0, p1, ut, vt, b0, w1t, b1):
    return pl.pallas_call(
        _node_mlp_body,
        grid=(_N // _BN,),
        in_specs=[
            pl.BlockSpec((_BN, _NF), lambda i: (i, 0)),
            pl.BlockSpec((_BN, _NF), lambda i: (i, 0)),
            pl.BlockSpec((_BN, _NF), lambda i: (i, 0)),
            pl.BlockSpec((_NF, _NF), lambda i: (0, 0)),
            pl.BlockSpec((_NF, _NF), lambda i: (0, 0)),
            pl.BlockSpec((1, _NF), lambda i: (0, 0)),
            pl.BlockSpec((_NF, _NF), lambda i: (0, 0)),
            pl.BlockSpec((1, _NF), lambda i: (0, 0)),
        ],
        out_specs=pl.BlockSpec((_BN, _NF), lambda i: (i, 0)),
        out_shape=jax.ShapeDtypeStruct((_N, _NF), jnp.float32),
    )(h, p0, p1, ut, vt, b0, w1t, b1)


def _coord_edge_body(c0_ref, geo_ref, ct_ref, w1t_ref, b1_ref, w2t_ref, out_ref):
    geo = geo_ref[...]
    ea = geo[:, 0:2]
    t0 = c0_ref[...].astype(jnp.float32) + jnp.dot(ea, ct_ref[...], preferred_element_type=jnp.float32)
    t0 = t0 * jax.nn.sigmoid(t0)
    t1 = jnp.dot(t0, w1t_ref[...], preferred_element_type=jnp.float32) + b1_ref[...]
    t1 = t1 * jax.nn.sigmoid(t1)
    tt = jnp.dot(t1, w2t_ref[...], preferred_element_type=jnp.float32)
    z = jnp.zeros((geo.shape[0], 13), jnp.float32)
    out_ref[...] = jnp.concatenate([geo[:, 2:5] * tt, z], axis=1)


def _tc_coord_edge(c0, geo, ct, w1t, b1, w2t):
    return pl.pallas_call(
        _coord_edge_body,
        grid=(_E // _BE,),
        in_specs=[
            pl.BlockSpec((_BE, _NF), lambda i: (i, 0)),
            pl.BlockSpec((_BE, 16), lambda i: (i, 0)),
            pl.BlockSpec((2, _NF), lambda i: (0, 0)),
            pl.BlockSpec((_NF, _NF), lambda i: (0, 0)),
            pl.BlockSpec((1, _NF), lambda i: (0, 0)),
            pl.BlockSpec((_NF, 1), lambda i: (0, 0)),
        ],
        out_specs=pl.BlockSpec((_BE, 16), lambda i: (i, 0)),
        out_shape=jax.ShapeDtypeStruct((_E, 16), jnp.float32),
    )(c0, geo, ct, w1t, b1, w2t)


def _coord_apply_body(x_ref, q0_ref, q1_ref, out_ref):
    q = (q0_ref[...].astype(jnp.float32)
         + q1_ref[...].astype(jnp.float32)) * (1.0 / _NORM)
    out_ref[...] = x_ref[...] + q[:, 0:3]


def _tc_coord_apply(x, q0, q1):
    return pl.pallas_call(
        _coord_apply_body,
        grid=(_N // _BN,),
        in_specs=[
            pl.BlockSpec((_BN, 3), lambda i: (i, 0)),
            pl.BlockSpec((_BN, 16), lambda i: (i, 0)),
            pl.BlockSpec((_BN, 16), lambda i: (i, 0)),
        ],
        out_specs=pl.BlockSpec((_BN, 3), lambda i: (i, 0)),
        out_shape=jax.ShapeDtypeStruct((_N, 3), jnp.float32),
    )(x, q0, q1)


def _pack_bf16(a):
    return jax.lax.bitcast_convert_type(
        a.reshape(a.shape[0], a.shape[1] // 2, 2), jnp.int32)


def _unpack_bf16(a):
    return jax.lax.bitcast_convert_type(a, jnp.bfloat16).reshape(a.shape[0], -1)


def kernel(h, x, edge_index, edge_attr, params):
    row = edge_index[0]
    col = edge_index[1]
    x16 = jnp.concatenate([x, jnp.zeros((_N, 13), jnp.float32)], axis=1)
    zeros128 = jnp.zeros((_RPT + 16, _NF), jnp.float32)
    zeros16 = jnp.zeros((_RPT + 16, 16), jnp.float32)

    xd = _sc_gather_combine(x16, x16, row, col, 16, -1, jnp.float32,
                            tct=False)
    geo = _tc_geom(xd, edge_attr)

    for i in range(2):
        w0 = params[f"gcl{i}_e_W0"]
        at = w0[:, :_NF].T
        bt = w0[:, _NF:2 * _NF].T
        ct = w0[:, 2 * _NF:].T
        ha, hb = _tc_node_pre(h, at, bt, params[f"gcl{i}_e_b0"][None, :])
        e0 = _sc_gather_combine(ha, hb, row, col, _NF, 1, jnp.float32)
        ef = _tc_edge_mlp(
            e0, geo, ct,
            params[f"gcl{i}_e_W1"].T,
            params[f"gcl{i}_e_b1"][None, :],
            params[f"gcl{i}_att_W"].T,
            params[f"gcl{i}_att_b"][None, :],
        )
        parts = _sc_scatter_add(ef, row, zeros128, _NF, jnp.float32)
        nw0 = params[f"gcl{i}_n_W0"]
        h = _tc_node_mlp(
            h, parts[0], parts[1],
            nw0[:, :_NF].T, nw0[:, _NF:].T,
            params[f"gcl{i}_n_b0"][None, :],
            params[f"gcl{i}_n_W1"].T,
            params[f"gcl{i}_n_b1"][None, :],
        )

    cw0 = params["c_W0"]
    ca, cb = _tc_node_pre(h, cw0[:, :_NF].T, cw0[:, _NF:2 * _NF].T,
                          params["c_b0"][None, :])
    c0 = _sc_gather_combine(ca, cb, row, col, _NF, 1, jnp.float32)
    trans = _tc_coord_edge(
        c0, geo, cw0[:, 2 * _NF:].T,
        params["c_W1"].T,
        params["c_b1"][None, :],
        params["c_W2"].T,
    )
    qparts = _sc_scatter_add(trans, row, zeros16, 16, jnp.float32,
                             tct=False)
    x_new = _tc_coord_apply(x, qparts[0], qparts[1])
    return h, x_new


# speedup vs baseline: 2.0817x; 1.0424x over previous
import functools

import jax
import jax.numpy as jnp
from jax import lax
from jax.experimental import pallas as pl
from jax.experimental.pallas import tpu as pltpu
from jax.experimental.pallas import tpu_sc as plsc

_N = 10000
_E = 320000
_NF = 128
_NORM = 100.0

_NC = 2
_NS = 16
_NW = _NC * _NS
_EPW = _E // _NW
_K = 80
_NCH = _EPW // _K
_RPT = 624


def _sc_gather_combine(ta, tb, row, col, D, sign, dtype, tct=True):
    bf16 = dtype == jnp.bfloat16
    W = D // 2 if bf16 else D
    G = D // 32 if bf16 else D // 16
    sdt = jnp.int32 if bf16 else jnp.float32
    mesh = plsc.VectorSubcoreMesh(core_axis_name="c", subcore_axis_name="s")

    @functools.partial(
        pl.kernel,
        out_type=jax.ShapeDtypeStruct((_E, W), sdt),
        mesh=mesh,
        scratch_types=[
            pltpu.VMEM((_K,), jnp.int32), pltpu.VMEM((_K,), jnp.int32),
            pltpu.VMEM((_K,), jnp.int32), pltpu.VMEM((_K,), jnp.int32),
            pltpu.VMEM((_K, W), sdt), pltpu.VMEM((_K, W), sdt),
            pltpu.VMEM((_K, W), sdt), pltpu.VMEM((_K, W), sdt),
            pltpu.SemaphoreType.DMA, pltpu.SemaphoreType.DMA,
        ],
        compiler_params=pltpu.CompilerParams(
            needs_layout_passes=not bf16, use_tc_tiling_on_sc=tct),
    )
    def kern(ta_h, tb_h, row_h, col_h, out_h,
             ir0, ir1, ic0, ic1, a0, a1, b0, b1, s0, s1):
        wid = lax.axis_index("c") * _NS + lax.axis_index("s")
        base = wid * _EPW
        irs = (ir0, ir1)
        ics = (ic0, ic1)
        abufs = (a0, a1)
        bbufs = (b0, b1)
        sems = (s0, s1)

        def fire(ci, t):
            off = base + ci * _K
            pltpu.sync_copy(row_h.at[pl.ds(off, _K)], irs[t])
            pltpu.sync_copy(col_h.at[pl.ds(off, _K)], ics[t])
            pltpu.async_copy(ta_h.at[irs[t]], abufs[t], sems[t])
            pltpu.async_copy(tb_h.at[ics[t]], bbufs[t], sems[t])

        def waitproc(ci, t):
            pltpu.make_async_copy(ta_h.at[irs[t]], abufs[t], sems[t]).wait()
            pltpu.make_async_copy(tb_h.at[ics[t]], bbufs[t], sems[t]).wait()
            a = abufs[t]
            b = bbufs[t]

            def addrow(r, carry):
                for g in range(G):
                    av = a[r, pl.ds(g * 16, 16)]
                    bv = b[r, pl.ds(g * 16, 16)]
                    if bf16:
                        av = plsc.bitcast(av, jnp.bfloat16)
                        bv = plsc.bitcast(bv, jnp.bfloat16)
                    res = av + bv if sign > 0 else av - bv
                    if bf16:
                        res = plsc.bitcast(res, jnp.int32)
                    a[r, pl.ds(g * 16, 16)] = res
                return carry

            lax.fori_loop(0, _K, addrow, 0)
            off = base + ci * _K
            pltpu.sync_copy(a, out_h.at[pl.ds(off, _K)])

        fire(0, 0)

        def body(i, carry):
            j = i * 2
            fire(j + 1, 1)
            waitproc(j, 0)
            fire(j + 2, 0)
            waitproc(j + 1, 1)
            return carry

        lax.fori_loop(0, (_NCH - 1) // 2, body, 0)
        waitproc(_NCH - 1, 0)

    return kern(ta, tb, row, col)


def _sc_gather_l0(ha, hb, xp, row, col):
    mesh = plsc.VectorSubcoreMesh(core_axis_name="c", subcore_axis_name="s")
    f32 = jnp.float32

    @functools.partial(
        pl.kernel,
        out_type=[jax.ShapeDtypeStruct((_E, _NF), f32),
                  jax.ShapeDtypeStruct((_E, _NF), f32)],
        mesh=mesh,
        scratch_types=[
            pltpu.VMEM((_K,), jnp.int32), pltpu.VMEM((_K,), jnp.int32),
            pltpu.VMEM((_K,), jnp.int32), pltpu.VMEM((_K,), jnp.int32),
            pltpu.VMEM((_K, _NF), f32), pltpu.VMEM((_K, _NF), f32),
            pltpu.VMEM((_K, _NF), f32), pltpu.VMEM((_K, _NF), f32),
            pltpu.VMEM((_K, _NF), f32), pltpu.VMEM((_K, _NF), f32),
            pltpu.VMEM((_K, _NF), f32), pltpu.VMEM((_K, _NF), f32),
            pltpu.SemaphoreType.DMA, pltpu.SemaphoreType.DMA,
        ],
    )
    def kern(ha_h, hb_h, xp_h, row_h, col_h, e0_h, xd_h,
             ir0, ir1, ic0, ic1, a0, a1, b0, b1, xa0, xa1, xb0, xb1, s0, s1):
        wid = lax.axis_index("c") * _NS + lax.axis_index("s")
        base = wid * _EPW
        irs = (ir0, ir1)
        ics = (ic0, ic1)
        abufs = (a0, a1)
        bbufs = (b0, b1)
        xabufs = (xa0, xa1)
        xbbufs = (xb0, xb1)
        sems = (s0, s1)

        def fire(ci, t):
            off = base + ci * _K
            pltpu.sync_copy(row_h.at[pl.ds(off, _K)], irs[t])
            pltpu.sync_copy(col_h.at[pl.ds(off, _K)], ics[t])
            pltpu.async_copy(ha_h.at[irs[t]], abufs[t], sems[t])
            pltpu.async_copy(hb_h.at[ics[t]], bbufs[t], sems[t])
            pltpu.async_copy(xp_h.at[irs[t]], xabufs[t], sems[t])
            pltpu.async_copy(xp_h.at[ics[t]], xbbufs[t], sems[t])

        def waitproc(ci, t):
            pltpu.make_async_copy(ha_h.at[irs[t]], abufs[t], sems[t]).wait()
            pltpu.make_async_copy(hb_h.at[ics[t]], bbufs[t], sems[t]).wait()
            pltpu.make_async_copy(xp_h.at[irs[t]], xabufs[t], sems[t]).wait()
            pltpu.make_async_copy(xp_h.at[ics[t]], xbbufs[t], sems[t]).wait()
            a = abufs[t]
            b = bbufs[t]
            xa = xabufs[t]
            xb = xbbufs[t]

            def addrow(r, carry):
                for g in range(_NF // 16):
                    sl = pl.ds(g * 16, 16)
                    a[r, sl] = a[r, sl] + b[r, sl]
                for g in range(2):
                    sl = pl.ds(g * 16, 16)
                    xa[r, sl] = xa[r, sl] - xb[r, sl]
                return carry

            lax.fori_loop(0, _K, addrow, 0)
            off = base + ci * _K
            pltpu.sync_copy(a, e0_h.at[pl.ds(off, _K)])
            pltpu.sync_copy(xa, xd_h.at[pl.ds(off, _K)])

        fire(0, 0)

        def body(i, carry):
            j = i * 2
            fire(j + 1, 1)
            waitproc(j, 0)
            fire(j + 2, 0)
            waitproc(j + 1, 1)
            return carry

        lax.fori_loop(0, (_NCH - 1) // 2, body, 0)
        waitproc(_NCH - 1, 0)

    return kern(ha, hb, xp, row, col)


def _sc_scatter_add(feat, row, zeros_tile, D, dtype, tct=True):
    mesh = plsc.VectorSubcoreMesh(core_axis_name="c", subcore_axis_name="s")

    @functools.partial(
        pl.kernel,
        out_type=jax.ShapeDtypeStruct((_NC, _N, D), dtype),
        mesh=mesh,
        scratch_types=[
            pltpu.VMEM((_K,), jnp.int32), pltpu.VMEM((_K,), jnp.int32),
            pltpu.VMEM((_K, D), dtype), pltpu.VMEM((_K, D), dtype),
            pltpu.VMEM_SHARED((_N, D), dtype),
            pltpu.SemaphoreType.DMA, pltpu.SemaphoreType.DMA,
        ],
        compiler_params=pltpu.CompilerParams(use_tc_tiling_on_sc=tct),
    )
    def kern(feat_h, row_h, z_h, out_h, i0, i1, f0, f1, acc, s0, s1):
        c = lax.axis_index("c")
        s = lax.axis_index("s")
        base = (c * _NS + s) * _EPW
        idxs = (i0, i1)
        fbufs = (f0, f1)
        sems = (s0, s1)

        pltpu.sync_copy(z_h.at[pl.ds(0, _RPT)], acc.at[pl.ds(s * _RPT, _RPT)])

        @pl.when(s == _NS - 1)
        def _():
            pltpu.sync_copy(z_h.at[pl.ds(_RPT, 16)],
                            acc.at[pl.ds(_NS * _RPT, 16)])

        plsc.subcore_barrier()

        def fire(ci, t):
            off = base + ci * _K
            pltpu.sync_copy(row_h.at[pl.ds(off, _K)], idxs[t])
            pltpu.async_copy(feat_h.at[pl.ds(off, _K)], fbufs[t], sems[t])

        def proc(ci, t):
            pltpu.make_async_copy(feat_h.at[pl.ds(0, _K)], fbufs[t], sems[t]).wait()
            pltpu.sync_copy(fbufs[t], acc.at[idxs[t]], add=True)

        fire(0, 0)

        def body(i, carry):
            j = i * 2
            fire(j + 1, 1)
            proc(j, 0)
            fire(j + 2, 0)
            proc(j + 1, 1)
            return carry

        lax.fori_loop(0, (_NCH - 1) // 2, body, 0)
        proc(_NCH - 1, 0)
        plsc.subcore_barrier()

        pltpu.sync_copy(acc.at[pl.ds(s * _RPT, _RPT)],
                        out_h.at[c, pl.ds(s * _RPT, _RPT)])

        @pl.when(s == _NS - 1)
        def _():
            pltpu.sync_copy(acc.at[pl.ds(_NS * _RPT, 16)],
                            out_h.at[c, pl.ds(_NS * _RPT, 16)])

    return kern(feat, row, zeros_tile)


_BN = 2000
_BE = 2000


def _node_pre_body(h_ref, at_ref, bt_ref, ba_ref, ha_ref, hb_ref):
    h = h_ref[...]
    ha_ref[...] = jnp.dot(h, at_ref[...], preferred_element_type=jnp.float32) + ba_ref[...]
    hb_ref[...] = jnp.dot(h, bt_ref[...], preferred_element_type=jnp.float32)


def _tc_node_pre(h, at, bt, ba):
    return pl.pallas_call(
        _node_pre_body,
        grid=(_N // _BN,),
        in_specs=[
            pl.BlockSpec((_BN, _NF), lambda i: (i, 0)),
            pl.BlockSpec((_NF, _NF), lambda i: (0, 0)),
            pl.BlockSpec((_NF, _NF), lambda i: (0, 0)),
            pl.BlockSpec((1, _NF), lambda i: (0, 0)),
        ],
        out_specs=[
            pl.BlockSpec((_BN, _NF), lambda i: (i, 0)),
            pl.BlockSpec((_BN, _NF), lambda i: (i, 0)),
        ],
        out_shape=[jax.ShapeDtypeStruct((_N, _NF), jnp.float32)] * 2,
    )(h, at, bt, ba)


def _geom_body(xd_ref, eat_ref, geo_ref):
    xd = xd_ref[...]
    r2 = jnp.sum(xd * xd, axis=1, keepdims=True)
    cd = xd / (jnp.sqrt(r2 + 1e-8) + 1.0)
    z = jnp.zeros((xd.shape[0], 11), jnp.float32)
    geo_ref[...] = jnp.concatenate([r2, eat_ref[...], cd[:, 0:3], z], axis=1)


def _tc_geom(xd, eattr):
    return pl.pallas_call(
        _geom_body,
        grid=(_E // _BE,),
        in_specs=[
            pl.BlockSpec((_BE, 16), lambda i: (i, 0)),
            pl.BlockSpec((_BE, 1), lambda i: (i, 0)),
        ],
        out_specs=pl.BlockSpec((_BE, 16), lambda i: (i, 0)),
        out_shape=jax.ShapeDtypeStruct((_E, 16), jnp.float32),
    )(xd, eattr)


def _edge_l0_body(xd_ref, eat_ref, e0_ref, ct_ref, w1t_ref, b1_ref,
                  aw_ref, ab_ref, out_ref, geo_ref):
    xd = xd_ref[...]
    r2 = jnp.sum(xd * xd, axis=1, keepdims=True)
    cd = xd / (jnp.sqrt(r2 + 1e-8) + 1.0)
    z = jnp.zeros((xd.shape[0], 11), jnp.float32)
    ea = jnp.concatenate([r2, eat_ref[...]], axis=1)
    geo_ref[...] = jnp.concatenate([ea, cd[:, 0:3], z], axis=1)
    t0 = e0_ref[...] + jnp.dot(ea, ct_ref[...], preferred_element_type=jnp.float32)
    t0 = t0 * jax.nn.sigmoid(t0)
    t1 = jnp.dot(t0, w1t_ref[...], preferred_element_type=jnp.float32) + b1_ref[...]
    t1 = t1 * jax.nn.sigmoid(t1)
    av = jnp.dot(t1, aw_ref[...], preferred_element_type=jnp.float32) + ab_ref[...]
    out_ref[...] = t1 * jax.nn.sigmoid(av)


def _tc_edge_l0(xd, eattr, e0, ct, w1t, b1, aw, ab):
    return pl.pallas_call(
        _edge_l0_body,
        grid=(_E // _BE,),
        in_specs=[
            pl.BlockSpec((_BE, _NF), lambda i: (i, 0)),
            pl.BlockSpec((_BE, 1), lambda i: (i, 0)),
            pl.BlockSpec((_BE, _NF), lambda i: (i, 0)),
            pl.BlockSpec((2, _NF), lambda i: (0, 0)),
            pl.BlockSpec((_NF, _NF), lambda i: (0, 0)),
            pl.BlockSpec((1, _NF), lambda i: (0, 0)),
            pl.BlockSpec((_NF, 1), lambda i: (0, 0)),
            pl.BlockSpec((1, 1), lambda i: (0, 0)),
        ],
        out_specs=[
            pl.BlockSpec((_BE, _NF), lambda i: (i, 0)),
            pl.BlockSpec((_BE, 16), lambda i: (i, 0)),
        ],
        out_shape=[jax.ShapeDtypeStruct((_E, _NF), jnp.float32),
                   jax.ShapeDtypeStruct((_E, 16), jnp.float32)],
    )(xd, eattr, e0, ct, w1t, b1, aw, ab)


def _node_mlp_fused_body(h_ref, p0_ref, p1_ref, ut_ref, vt_ref, b0_ref,
                         w1t_ref, b1_ref, at2_ref, bt2_ref, ba2_ref,
                         out_ref, ha2_ref, hb2_ref):
    h = h_ref[...]
    agg = (p0_ref[...].astype(jnp.float32)
           + p1_ref[...].astype(jnp.float32)) * (1.0 / _NORM)
    t = (jnp.dot(h, ut_ref[...], preferred_element_type=jnp.float32)
         + jnp.dot(agg, vt_ref[...], preferred_element_type=jnp.float32)
         + b0_ref[...])
    t = t * jax.nn.sigmoid(t)
    dh = jnp.dot(t, w1t_ref[...], preferred_element_type=jnp.float32) + b1_ref[...]
    hn = h + dh
    out_ref[...] = hn
    ha2_ref[...] = jnp.dot(hn, at2_ref[...], preferred_element_type=jnp.float32) + ba2_ref[...]
    hb2_ref[...] = jnp.dot(hn, bt2_ref[...], preferred_element_type=jnp.float32)


def _tc_node_mlp_fused(h, p0, p1, ut, vt, b0, w1t, b1, at2, bt2, ba2):
    return pl.pallas_call(
        _node_mlp_fused_body,
        grid=(_N // _BN,),
        in_specs=[
            pl.BlockSpec((_BN, _NF), lambda i: (i, 0)),
            pl.BlockSpec((_BN, _NF), lambda i: (i, 0)),
            pl.BlockSpec((_BN, _NF), lambda i: (i, 0)),
            pl.BlockSpec((_NF, _NF), lambda i: (0, 0)),
            pl.BlockSpec((_NF, _NF), lambda i: (0, 0)),
            pl.BlockSpec((1, _NF), lambda i: (0, 0)),
            pl.BlockSpec((_NF, _NF), lambda i: (0, 0)),
            pl.BlockSpec((1, _NF), lambda i: (0, 0)),
            pl.BlockSpec((_NF, _NF), lambda i: (0, 0)),
            pl.BlockSpec((_NF, _NF), lambda i: (0, 0)),
            pl.BlockSpec((1, _NF), lambda i: (0, 0)),
        ],
        out_specs=[pl.BlockSpec((_BN, _NF), lambda i: (i, 0))] * 3,
        out_shape=[jax.ShapeDtypeStruct((_N, _NF), jnp.float32)] * 3,
    )(h, p0, p1, ut, vt, b0, w1t, b1, at2, bt2, ba2)


def _edge_mlp_body(e0_ref, geo_ref, ct_ref, w1t_ref, b1_ref, aw_ref, ab_ref, out_ref):
    e0 = e0_ref[...].astype(jnp.float32)
    ea = geo_ref[...][:, 0:2]
    t0 = e0 + jnp.dot(ea, ct_ref[...], preferred_element_type=jnp.float32)
    t0 = t0 * jax.nn.sigmoid(t0)
    t1 = jnp.dot(t0, w1t_ref[...], preferred_element_type=jnp.float32) + b1_ref[...]
    t1 = t1 * jax.nn.sigmoid(t1)
    av = jnp.dot(t1, aw_ref[...], preferred_element_type=jnp.float32) + ab_ref[...]
    out_ref[...] = t1 * jax.nn.sigmoid(av)


def _tc_edge_mlp(e0, geo, ct, w1t, b1, aw, ab):
    return pl.pallas_call(
        _edge_mlp_body,
        grid=(_E // _BE,),
        in_specs=[
            pl.BlockSpec((_BE, _NF), lambda i: (i, 0)),
            pl.BlockSpec((_BE, 16), lambda i: (i, 0)),
            pl.BlockSpec((2, _NF), lambda i: (0, 0)),
            pl.BlockSpec((_NF, _NF), lambda i: (0, 0)),
            pl.BlockSpec((1, _NF), lambda i: (0, 0)),
            pl.BlockSpec((_NF, 1), lambda i: (0, 0)),
            pl.BlockSpec((1, 1), lambda i: (0, 0)),
        ],
        out_specs=pl.BlockSpec((_BE, _NF), lambda i: (i, 0)),
        out_shape=jax.ShapeDtypeStruct((_E, _NF), jnp.float32),
    )(e0, geo, ct, w1t, b1, aw, ab)


def _node_mlp_body(h_ref, p0_ref, p1_ref, ut_ref, vt_ref, b0_ref, w1t_ref, b1_ref, out_ref):
    h = h_ref[...]
    agg = (p0_ref[...].astype(jnp.float32)
           + p1_ref[...].astype(jnp.float32)) * (1.0 / _NORM)
    t = (jnp.dot(h, ut_ref[...], preferred_element_type=jnp.float32)
         + jnp.dot(agg, vt_ref[...], preferred_element_type=jnp.float32)
         + b0_ref[...])
    t = t * jax.nn.sigmoid(t)
    dh = jnp.dot(t, w1t_ref[...], preferred_element_type=jnp.float32) + b1_ref[...]
    out_ref[...] = h + dh


def _tc_node_mlp(h, p0, p1, ut, vt, b0, w1t, b1):
    return pl.pallas_call(
        _node_mlp_body,
        grid=(_N // _BN,),
        in_specs=[
            pl.BlockSpec((_BN, _NF), lambda i: (i, 0)),
            pl.BlockSpec((_BN, _NF), lambda i: (i, 0)),
            pl.BlockSpec((_BN, _NF), lambda i: (i, 0)),
            pl.BlockSpec((_NF, _NF), lambda i: (0, 0)),
            pl.BlockSpec((_NF, _NF), lambda i: (0, 0)),
            pl.BlockSpec((1, _NF), lambda i: (0, 0)),
            pl.BlockSpec((_NF, _NF), lambda i: (0, 0)),
            pl.BlockSpec((1, _NF), lambda i: (0, 0)),
        ],
        out_specs=pl.BlockSpec((_BN, _NF), lambda i: (i, 0)),
        out_shape=jax.ShapeDtypeStruct((_N, _NF), jnp.float32),
    )(h, p0, p1, ut, vt, b0, w1t, b1)


def _coord_edge_body(c0_ref, geo_ref, ct_ref, w1t_ref, b1_ref, w2t_ref, out_ref):
    geo = geo_ref[...]
    ea = geo[:, 0:2]
    t0 = c0_ref[...].astype(jnp.float32) + jnp.dot(ea, ct_ref[...], preferred_element_type=jnp.float32)
    t0 = t0 * jax.nn.sigmoid(t0)
    t1 = jnp.dot(t0, w1t_ref[...], preferred_element_type=jnp.float32) + b1_ref[...]
    t1 = t1 * jax.nn.sigmoid(t1)
    tt = jnp.dot(t1, w2t_ref[...], preferred_element_type=jnp.float32)
    z = jnp.zeros((geo.shape[0], 13), jnp.float32)
    out_ref[...] = jnp.concatenate([geo[:, 2:5] * tt, z], axis=1)


def _tc_coord_edge(c0, geo, ct, w1t, b1, w2t):
    return pl.pallas_call(
        _coord_edge_body,
        grid=(_E // _BE,),
        in_specs=[
            pl.BlockSpec((_BE, _NF), lambda i: (i, 0)),
            pl.BlockSpec((_BE, 16), lambda i: (i, 0)),
            pl.BlockSpec((2, _NF), lambda i: (0, 0)),
            pl.BlockSpec((_NF, _NF), lambda i: (0, 0)),
            pl.BlockSpec((1, _NF), lambda i: (0, 0)),
            pl.BlockSpec((_NF, 1), lambda i: (0, 0)),
        ],
        out_specs=pl.BlockSpec((_BE, 16), lambda i: (i, 0)),
        out_shape=jax.ShapeDtypeStruct((_E, 16), jnp.float32),
    )(c0, geo, ct, w1t, b1, w2t)


def _coord_apply_body(x_ref, q0_ref, q1_ref, out_ref):
    q = (q0_ref[...].astype(jnp.float32)
         + q1_ref[...].astype(jnp.float32)) * (1.0 / _NORM)
    out_ref[...] = x_ref[...] + q[:, 0:3]


def _tc_coord_apply(x, q0, q1):
    return pl.pallas_call(
        _coord_apply_body,
        grid=(_N // _BN,),
        in_specs=[
            pl.BlockSpec((_BN, 3), lambda i: (i, 0)),
            pl.BlockSpec((_BN, 16), lambda i: (i, 0)),
            pl.BlockSpec((_BN, 16), lambda i: (i, 0)),
        ],
        out_specs=pl.BlockSpec((_BN, 3), lambda i: (i, 0)),
        out_shape=jax.ShapeDtypeStruct((_N, 3), jnp.float32),
    )(x, q0, q1)


def _pack_bf16(a):
    return jax.lax.bitcast_convert_type(
        a.reshape(a.shape[0], a.shape[1] // 2, 2), jnp.int32)


def _unpack_bf16(a):
    return jax.lax.bitcast_convert_type(a, jnp.bfloat16).reshape(a.shape[0], -1)


def kernel(h, x, edge_index, edge_attr, params):
    row = edge_index[0]
    col = edge_index[1]
    x128 = jnp.concatenate([x, jnp.zeros((_N, _NF - 3), jnp.float32)], axis=1)
    zeros128 = jnp.zeros((_RPT + 16, _NF), jnp.float32)
    zeros16 = jnp.zeros((_RPT + 16, 16), jnp.float32)

    def ew(i):
        w0 = params[f"gcl{i}_e_W0"]
        return (w0[:, :_NF].T, w0[:, _NF:2 * _NF].T, w0[:, 2 * _NF:].T,
                params[f"gcl{i}_e_b0"][None, :])

    at0, bt0, ct0, ba0 = ew(0)
    at1, bt1, ct1, ba1 = ew(1)
    cw0 = params["c_W0"]
    cat, cbt, cct, cba = (cw0[:, :_NF].T, cw0[:, _NF:2 * _NF].T,
                          cw0[:, 2 * _NF:].T, params["c_b0"][None, :])

    ha0, hb0 = _tc_node_pre(h, at0, bt0, ba0)
    e0, xd = _sc_gather_l0(ha0, hb0, x128, row, col)
    ef0, geo = _tc_edge_l0(
        xd, edge_attr, e0, ct0,
        params["gcl0_e_W1"].T, params["gcl0_e_b1"][None, :],
        params["gcl0_att_W"].T, params["gcl0_att_b"][None, :])
    parts0 = _sc_scatter_add(ef0, row, zeros128, _NF, jnp.float32)
    nw0 = params["gcl0_n_W0"]
    h1, ha1, hb1 = _tc_node_mlp_fused(
        h, parts0[0], parts0[1],
        nw0[:, :_NF].T, nw0[:, _NF:].T, params["gcl0_n_b0"][None, :],
        params["gcl0_n_W1"].T, params["gcl0_n_b1"][None, :],
        at1, bt1, ba1)

    e1 = _sc_gather_combine(ha1, hb1, row, col, _NF, 1, jnp.float32)
    ef1 = _tc_edge_mlp(
        e1, geo, ct1,
        params["gcl1_e_W1"].T, params["gcl1_e_b1"][None, :],
        params["gcl1_att_W"].T, params["gcl1_att_b"][None, :])
    parts1 = _sc_scatter_add(ef1, row, zeros128, _NF, jnp.float32)
    nw1 = params["gcl1_n_W0"]
    h2, ca, cb = _tc_node_mlp_fused(
        h1, parts1[0], parts1[1],
        nw1[:, :_NF].T, nw1[:, _NF:].T, params["gcl1_n_b0"][None, :],
        params["gcl1_n_W1"].T, params["gcl1_n_b1"][None, :],
        cat, cbt, cba)

    c0 = _sc_gather_combine(ca, cb, row, col, _NF, 1, jnp.float32)
    trans = _tc_coord_edge(
        c0, geo, cct,
        params["c_W1"].T,
        params["c_b1"][None, :],
        params["c_W2"].T,
    )
    qparts = _sc_scatter_add(trans, row, zeros16, 16, jnp.float32,
                             tct=False)
    x_new = _tc_coord_apply(x, qparts[0], qparts[1])
    return h2, x_new


# speedup vs baseline: 2.0839x; 1.0011x over previous
import functools

import jax
import jax.numpy as jnp
from jax import lax
from jax.experimental import pallas as pl
from jax.experimental.pallas import tpu as pltpu
from jax.experimental.pallas import tpu_sc as plsc

_N = 10000
_E = 320000
_NF = 128
_NORM = 100.0

_NC = 2
_NS = 16
_NW = _NC * _NS
_EPW = _E // _NW
_K = 80
_NCH = _EPW // _K
_RPT = 624


def _sc_gather_combine(ta, tb, row, col, D, sign, dtype, tct=True):
    bf16 = dtype == jnp.bfloat16
    W = D // 2 if bf16 else D
    G = D // 32 if bf16 else D // 16
    sdt = jnp.int32 if bf16 else jnp.float32
    mesh = plsc.VectorSubcoreMesh(core_axis_name="c", subcore_axis_name="s")

    @functools.partial(
        pl.kernel,
        out_type=jax.ShapeDtypeStruct((_E, W), sdt),
        mesh=mesh,
        scratch_types=[
            pltpu.VMEM((_K,), jnp.int32), pltpu.VMEM((_K,), jnp.int32),
            pltpu.VMEM((_K,), jnp.int32), pltpu.VMEM((_K,), jnp.int32),
            pltpu.VMEM((_K, W), sdt), pltpu.VMEM((_K, W), sdt),
            pltpu.VMEM((_K, W), sdt), pltpu.VMEM((_K, W), sdt),
            pltpu.SemaphoreType.DMA, pltpu.SemaphoreType.DMA,
            pltpu.SemaphoreType.DMA, pltpu.SemaphoreType.DMA,
        ],
        compiler_params=pltpu.CompilerParams(
            needs_layout_passes=not bf16, use_tc_tiling_on_sc=tct),
    )
    def kern(ta_h, tb_h, row_h, col_h, out_h,
             ir0, ir1, ic0, ic1, a0, a1, b0, b1, s0, s1, w0, w1):
        wid = lax.axis_index("c") * _NS + lax.axis_index("s")
        base = wid * _EPW
        irs = (ir0, ir1)
        ics = (ic0, ic1)
        abufs = (a0, a1)
        bbufs = (b0, b1)
        sems = (s0, s1)
        wsems = (w0, w1)

        def fire(ci, t):
            off = base + ci * _K

            @pl.when(ci >= 2)
            def _():
                pltpu.make_async_copy(out_h.at[pl.ds(base, _K)],
                                      abufs[t], wsems[t]).wait()

            pltpu.sync_copy(row_h.at[pl.ds(off, _K)], irs[t])
            pltpu.sync_copy(col_h.at[pl.ds(off, _K)], ics[t])
            pltpu.async_copy(ta_h.at[irs[t]], abufs[t], sems[t])
            pltpu.async_copy(tb_h.at[ics[t]], bbufs[t], sems[t])

        def waitproc(ci, t):
            pltpu.make_async_copy(ta_h.at[irs[t]], abufs[t], sems[t]).wait()
            pltpu.make_async_copy(tb_h.at[ics[t]], bbufs[t], sems[t]).wait()
            a = abufs[t]
            b = bbufs[t]

            def addrow(r, carry):
                for g in range(G):
                    av = a[r, pl.ds(g * 16, 16)]
                    bv = b[r, pl.ds(g * 16, 16)]
                    if bf16:
                        av = plsc.bitcast(av, jnp.bfloat16)
                        bv = plsc.bitcast(bv, jnp.bfloat16)
                    res = av + bv if sign > 0 else av - bv
                    if bf16:
                        res = plsc.bitcast(res, jnp.int32)
                    a[r, pl.ds(g * 16, 16)] = res
                return carry

            lax.fori_loop(0, _K, addrow, 0)
            off = base + ci * _K
            pltpu.async_copy(a, out_h.at[pl.ds(off, _K)], wsems[t])

        fire(0, 0)

        def body(i, carry):
            j = i * 2
            fire(j + 1, 1)
            waitproc(j, 0)
            fire(j + 2, 0)
            waitproc(j + 1, 1)
            return carry

        lax.fori_loop(0, (_NCH - 1) // 2, body, 0)
        waitproc(_NCH - 1, 0)
        for t in range(2):
            pltpu.make_async_copy(out_h.at[pl.ds(base, _K)],
                                  abufs[t], wsems[t]).wait()

    return kern(ta, tb, row, col)


def _sc_gather_l0(ha, hb, xp, row, col):
    mesh = plsc.VectorSubcoreMesh(core_axis_name="c", subcore_axis_name="s")
    f32 = jnp.float32

    @functools.partial(
        pl.kernel,
        out_type=[jax.ShapeDtypeStruct((_E, _NF), f32),
                  jax.ShapeDtypeStruct((_E, _NF), f32)],
        mesh=mesh,
        scratch_types=[
            pltpu.VMEM((_K,), jnp.int32), pltpu.VMEM((_K,), jnp.int32),
            pltpu.VMEM((_K,), jnp.int32), pltpu.VMEM((_K,), jnp.int32),
            pltpu.VMEM((_K, _NF), f32), pltpu.VMEM((_K, _NF), f32),
            pltpu.VMEM((_K, _NF), f32), pltpu.VMEM((_K, _NF), f32),
            pltpu.VMEM((_K, _NF), f32), pltpu.VMEM((_K, _NF), f32),
            pltpu.VMEM((_K, _NF), f32), pltpu.VMEM((_K, _NF), f32),
            pltpu.SemaphoreType.DMA, pltpu.SemaphoreType.DMA,
            pltpu.SemaphoreType.DMA, pltpu.SemaphoreType.DMA,
        ],
    )
    def kern(ha_h, hb_h, xp_h, row_h, col_h, e0_h, xd_h,
             ir0, ir1, ic0, ic1, a0, a1, b0, b1, xa0, xa1, xb0, xb1, s0, s1,
             w0, w1):
        wid = lax.axis_index("c") * _NS + lax.axis_index("s")
        base = wid * _EPW
        irs = (ir0, ir1)
        ics = (ic0, ic1)
        abufs = (a0, a1)
        bbufs = (b0, b1)
        xabufs = (xa0, xa1)
        xbbufs = (xb0, xb1)
        sems = (s0, s1)
        wsems = (w0, w1)

        def fire(ci, t):
            off = base + ci * _K

            @pl.when(ci >= 2)
            def _():
                pltpu.make_async_copy(e0_h.at[pl.ds(base, _K)],
                                      abufs[t], wsems[t]).wait()
                pltpu.make_async_copy(xd_h.at[pl.ds(base, _K)],
                                      xabufs[t], wsems[t]).wait()

            pltpu.sync_copy(row_h.at[pl.ds(off, _K)], irs[t])
            pltpu.sync_copy(col_h.at[pl.ds(off, _K)], ics[t])
            pltpu.async_copy(ha_h.at[irs[t]], abufs[t], sems[t])
            pltpu.async_copy(hb_h.at[ics[t]], bbufs[t], sems[t])
            pltpu.async_copy(xp_h.at[irs[t]], xabufs[t], sems[t])
            pltpu.async_copy(xp_h.at[ics[t]], xbbufs[t], sems[t])

        def waitproc(ci, t):
            pltpu.make_async_copy(ha_h.at[irs[t]], abufs[t], sems[t]).wait()
            pltpu.make_async_copy(hb_h.at[ics[t]], bbufs[t], sems[t]).wait()
            pltpu.make_async_copy(xp_h.at[irs[t]], xabufs[t], sems[t]).wait()
            pltpu.make_async_copy(xp_h.at[ics[t]], xbbufs[t], sems[t]).wait()
            a = abufs[t]
            b = bbufs[t]
            xa = xabufs[t]
            xb = xbbufs[t]

            def addrow(r, carry):
                for g in range(_NF // 16):
                    sl = pl.ds(g * 16, 16)
                    a[r, sl] = a[r, sl] + b[r, sl]
                for g in range(2):
                    sl = pl.ds(g * 16, 16)
                    xa[r, sl] = xa[r, sl] - xb[r, sl]
                return carry

            lax.fori_loop(0, _K, addrow, 0)
            off = base + ci * _K
            pltpu.async_copy(a, e0_h.at[pl.ds(off, _K)], wsems[t])
            pltpu.async_copy(xa, xd_h.at[pl.ds(off, _K)], wsems[t])

        fire(0, 0)

        def body(i, carry):
            j = i * 2
            fire(j + 1, 1)
            waitproc(j, 0)
            fire(j + 2, 0)
            waitproc(j + 1, 1)
            return carry

        lax.fori_loop(0, (_NCH - 1) // 2, body, 0)
        waitproc(_NCH - 1, 0)
        for t in range(2):
            pltpu.make_async_copy(e0_h.at[pl.ds(base, _K)],
                                  abufs[t], wsems[t]).wait()
            pltpu.make_async_copy(xd_h.at[pl.ds(base, _K)],
                                  xabufs[t], wsems[t]).wait()

    return kern(ha, hb, xp, row, col)


def _sc_scatter_add(feat, row, zeros_tile, D, dtype, tct=True):
    mesh = plsc.VectorSubcoreMesh(core_axis_name="c", subcore_axis_name="s")

    @functools.partial(
        pl.kernel,
        out_type=jax.ShapeDtypeStruct((_NC, _N, D), dtype),
        mesh=mesh,
        scratch_types=[
            pltpu.VMEM((_K,), jnp.int32), pltpu.VMEM((_K,), jnp.int32),
            pltpu.VMEM((_K, D), dtype), pltpu.VMEM((_K, D), dtype),
            pltpu.VMEM_SHARED((_N, D), dtype),
            pltpu.SemaphoreType.DMA, pltpu.SemaphoreType.DMA,
            pltpu.SemaphoreType.DMA, pltpu.SemaphoreType.DMA,
        ],
        compiler_params=pltpu.CompilerParams(use_tc_tiling_on_sc=tct),
    )
    def kern(feat_h, row_h, z_h, out_h, i0, i1, f0, f1, acc, s0, s1, w0, w1):
        c = lax.axis_index("c")
        s = lax.axis_index("s")
        base = (c * _NS + s) * _EPW
        idxs = (i0, i1)
        fbufs = (f0, f1)
        sems = (s0, s1)
        wsems = (w0, w1)

        pltpu.sync_copy(z_h.at[pl.ds(0, _RPT)], acc.at[pl.ds(s * _RPT, _RPT)])

        @pl.when(s == _NS - 1)
        def _():
            pltpu.sync_copy(z_h.at[pl.ds(_RPT, 16)],
                            acc.at[pl.ds(_NS * _RPT, 16)])

        plsc.subcore_barrier()

        def fire(ci, t):
            off = base + ci * _K

            @pl.when(ci >= 2)
            def _():
                pltpu.make_async_copy(feat_h.at[pl.ds(base, _K)],
                                      fbufs[t], wsems[t]).wait()

            pltpu.sync_copy(row_h.at[pl.ds(off, _K)], idxs[t])
            pltpu.async_copy(feat_h.at[pl.ds(off, _K)], fbufs[t], sems[t])

        def proc(ci, t):
            pltpu.make_async_copy(feat_h.at[pl.ds(0, _K)], fbufs[t], sems[t]).wait()
            pltpu.async_copy(fbufs[t], acc.at[idxs[t]], wsems[t], add=True)

        fire(0, 0)

        def body(i, carry):
            j = i * 2
            fire(j + 1, 1)
            proc(j, 0)
            fire(j + 2, 0)
            proc(j + 1, 1)
            return carry

        lax.fori_loop(0, (_NCH - 1) // 2, body, 0)
        proc(_NCH - 1, 0)
        for t in range(2):
            pltpu.make_async_copy(feat_h.at[pl.ds(base, _K)],
                                  fbufs[t], wsems[t]).wait()
        plsc.subcore_barrier()

        pltpu.sync_copy(acc.at[pl.ds(s * _RPT, _RPT)],
                        out_h.at[c, pl.ds(s * _RPT, _RPT)])

        @pl.when(s == _NS - 1)
        def _():
            pltpu.sync_copy(acc.at[pl.ds(_NS * _RPT, 16)],
                            out_h.at[c, pl.ds(_NS * _RPT, 16)])

    return kern(feat, row, zeros_tile)


_BN = 2000
_BE = 2000


def _node_pre_body(h_ref, at_ref, bt_ref, ba_ref, ha_ref, hb_ref):
    h = h_ref[...]
    ha_ref[...] = jnp.dot(h, at_ref[...], preferred_element_type=jnp.float32) + ba_ref[...]
    hb_ref[...] = jnp.dot(h, bt_ref[...], preferred_element_type=jnp.float32)


def _tc_node_pre(h, at, bt, ba):
    return pl.pallas_call(
        _node_pre_body,
        grid=(_N // _BN,),
        in_specs=[
            pl.BlockSpec((_BN, _NF), lambda i: (i, 0)),
            pl.BlockSpec((_NF, _NF), lambda i: (0, 0)),
            pl.BlockSpec((_NF, _NF), lambda i: (0, 0)),
            pl.BlockSpec((1, _NF), lambda i: (0, 0)),
        ],
        out_specs=[
            pl.BlockSpec((_BN, _NF), lambda i: (i, 0)),
            pl.BlockSpec((_BN, _NF), lambda i: (i, 0)),
        ],
        out_shape=[jax.ShapeDtypeStruct((_N, _NF), jnp.float32)] * 2,
    )(h, at, bt, ba)


def _geom_body(xd_ref, eat_ref, geo_ref):
    xd = xd_ref[...]
    r2 = jnp.sum(xd * xd, axis=1, keepdims=True)
    cd = xd / (jnp.sqrt(r2 + 1e-8) + 1.0)
    z = jnp.zeros((xd.shape[0], 11), jnp.float32)
    geo_ref[...] = jnp.concatenate([r2, eat_ref[...], cd[:, 0:3], z], axis=1)


def _tc_geom(xd, eattr):
    return pl.pallas_call(
        _geom_body,
        grid=(_E // _BE,),
        in_specs=[
            pl.BlockSpec((_BE, 16), lambda i: (i, 0)),
            pl.BlockSpec((_BE, 1), lambda i: (i, 0)),
        ],
        out_specs=pl.BlockSpec((_BE, 16), lambda i: (i, 0)),
        out_shape=jax.ShapeDtypeStruct((_E, 16), jnp.float32),
    )(xd, eattr)


def _edge_l0_body(xd_ref, eat_ref, e0_ref, ct_ref, w1t_ref, b1_ref,
                  aw_ref, ab_ref, out_ref, geo_ref):
    xd = xd_ref[...]
    r2 = jnp.sum(xd * xd, axis=1, keepdims=True)
    cd = xd / (jnp.sqrt(r2 + 1e-8) + 1.0)
    z = jnp.zeros((xd.shape[0], 11), jnp.float32)
    ea = jnp.concatenate([r2, eat_ref[...]], axis=1)
    geo_ref[...] = jnp.concatenate([ea, cd[:, 0:3], z], axis=1)
    t0 = e0_ref[...] + jnp.dot(ea, ct_ref[...], preferred_element_type=jnp.float32)
    t0 = t0 * jax.nn.sigmoid(t0)
    t1 = jnp.dot(t0, w1t_ref[...], preferred_element_type=jnp.float32) + b1_ref[...]
    t1 = t1 * jax.nn.sigmoid(t1)
    av = jnp.dot(t1, aw_ref[...], preferred_element_type=jnp.float32) + ab_ref[...]
    out_ref[...] = t1 * jax.nn.sigmoid(av)


def _tc_edge_l0(xd, eattr, e0, ct, w1t, b1, aw, ab):
    return pl.pallas_call(
        _edge_l0_body,
        grid=(_E // _BE,),
        in_specs=[
            pl.BlockSpec((_BE, _NF), lambda i: (i, 0)),
            pl.BlockSpec((_BE, 1), lambda i: (i, 0)),
            pl.BlockSpec((_BE, _NF), lambda i: (i, 0)),
            pl.BlockSpec((2, _NF), lambda i: (0, 0)),
            pl.BlockSpec((_NF, _NF), lambda i: (0, 0)),
            pl.BlockSpec((1, _NF), lambda i: (0, 0)),
            pl.BlockSpec((_NF, 1), lambda i: (0, 0)),
            pl.BlockSpec((1, 1), lambda i: (0, 0)),
        ],
        out_specs=[
            pl.BlockSpec((_BE, _NF), lambda i: (i, 0)),
            pl.BlockSpec((_BE, 16), lambda i: (i, 0)),
        ],
        out_shape=[jax.ShapeDtypeStruct((_E, _NF), jnp.float32),
                   jax.ShapeDtypeStruct((_E, 16), jnp.float32)],
    )(xd, eattr, e0, ct, w1t, b1, aw, ab)


def _node_mlp_fused_body(h_ref, p0_ref, p1_ref, ut_ref, vt_ref, b0_ref,
                         w1t_ref, b1_ref, at2_ref, bt2_ref, ba2_ref,
                         out_ref, ha2_ref, hb2_ref):
    h = h_ref[...]
    agg = (p0_ref[...].astype(jnp.float32)
           + p1_ref[...].astype(jnp.float32)) * (1.0 / _NORM)
    t = (jnp.dot(h, ut_ref[...], preferred_element_type=jnp.float32)
         + jnp.dot(agg, vt_ref[...], preferred_element_type=jnp.float32)
         + b0_ref[...])
    t = t * jax.nn.sigmoid(t)
    dh = jnp.dot(t, w1t_ref[...], preferred_element_type=jnp.float32) + b1_ref[...]
    hn = h + dh
    out_ref[...] = hn
    ha2_ref[...] = jnp.dot(hn, at2_ref[...], preferred_element_type=jnp.float32) + ba2_ref[...]
    hb2_ref[...] = jnp.dot(hn, bt2_ref[...], preferred_element_type=jnp.float32)


def _tc_node_mlp_fused(h, p0, p1, ut, vt, b0, w1t, b1, at2, bt2, ba2):
    return pl.pallas_call(
        _node_mlp_fused_body,
        grid=(_N // _BN,),
        in_specs=[
            pl.BlockSpec((_BN, _NF), lambda i: (i, 0)),
            pl.BlockSpec((_BN, _NF), lambda i: (i, 0)),
            pl.BlockSpec((_BN, _NF), lambda i: (i, 0)),
            pl.BlockSpec((_NF, _NF), lambda i: (0, 0)),
            pl.BlockSpec((_NF, _NF), lambda i: (0, 0)),
            pl.BlockSpec((1, _NF), lambda i: (0, 0)),
            pl.BlockSpec((_NF, _NF), lambda i: (0, 0)),
            pl.BlockSpec((1, _NF), lambda i: (0, 0)),
            pl.BlockSpec((_NF, _NF), lambda i: (0, 0)),
            pl.BlockSpec((_NF, _NF), lambda i: (0, 0)),
            pl.BlockSpec((1, _NF), lambda i: (0, 0)),
        ],
        out_specs=[pl.BlockSpec((_BN, _NF), lambda i: (i, 0))] * 3,
        out_shape=[jax.ShapeDtypeStruct((_N, _NF), jnp.float32)] * 3,
    )(h, p0, p1, ut, vt, b0, w1t, b1, at2, bt2, ba2)


def _edge_mlp_body(e0_ref, geo_ref, ct_ref, w1t_ref, b1_ref, aw_ref, ab_ref, out_ref):
    e0 = e0_ref[...].astype(jnp.float32)
    ea = geo_ref[...][:, 0:2]
    t0 = e0 + jnp.dot(ea, ct_ref[...], preferred_element_type=jnp.float32)
    t0 = t0 * jax.nn.sigmoid(t0)
    t1 = jnp.dot(t0, w1t_ref[...], preferred_element_type=jnp.float32) + b1_ref[...]
    t1 = t1 * jax.nn.sigmoid(t1)
    av = jnp.dot(t1, aw_ref[...], preferred_element_type=jnp.float32) + ab_ref[...]
    out_ref[...] = t1 * jax.nn.sigmoid(av)


def _tc_edge_mlp(e0, geo, ct, w1t, b1, aw, ab):
    return pl.pallas_call(
        _edge_mlp_body,
        grid=(_E // _BE,),
        in_specs=[
            pl.BlockSpec((_BE, _NF), lambda i: (i, 0)),
            pl.BlockSpec((_BE, 16), lambda i: (i, 0)),
            pl.BlockSpec((2, _NF), lambda i: (0, 0)),
            pl.BlockSpec((_NF, _NF), lambda i: (0, 0)),
            pl.BlockSpec((1, _NF), lambda i: (0, 0)),
            pl.BlockSpec((_NF, 1), lambda i: (0, 0)),
            pl.BlockSpec((1, 1), lambda i: (0, 0)),
        ],
        out_specs=pl.BlockSpec((_BE, _NF), lambda i: (i, 0)),
        out_shape=jax.ShapeDtypeStruct((_E, _NF), jnp.float32),
    )(e0, geo, ct, w1t, b1, aw, ab)


def _node_mlp_body(h_ref, p0_ref, p1_ref, ut_ref, vt_ref, b0_ref, w1t_ref, b1_ref, out_ref):
    h = h_ref[...]
    agg = (p0_ref[...].astype(jnp.float32)
           + p1_ref[...].astype(jnp.float32)) * (1.0 / _NORM)
    t = (jnp.dot(h, ut_ref[...], preferred_element_type=jnp.float32)
         + jnp.dot(agg, vt_ref[...], preferred_element_type=jnp.float32)
         + b0_ref[...])
    t = t * jax.nn.sigmoid(t)
    dh = jnp.dot(t, w1t_ref[...], preferred_element_type=jnp.float32) + b1_ref[...]
    out_ref[...] = h + dh


def _tc_node_mlp(h, p0, p1, ut, vt, b0, w1t, b1):
    return pl.pallas_call(
        _node_mlp_body,
        grid=(_N // _BN,),
        in_specs=[
            pl.BlockSpec((_BN, _NF), lambda i: (i, 0)),
            pl.BlockSpec((_BN, _NF), lambda i: (i, 0)),
            pl.BlockSpec((_BN, _NF), lambda i: (i, 0)),
            pl.BlockSpec((_NF, _NF), lambda i: (0, 0)),
            pl.BlockSpec((_NF, _NF), lambda i: (0, 0)),
            pl.BlockSpec((1, _NF), lambda i: (0, 0)),
            pl.BlockSpec((_NF, _NF), lambda i: (0, 0)),
            pl.BlockSpec((1, _NF), lambda i: (0, 0)),
        ],
        out_specs=pl.BlockSpec((_BN, _NF), lambda i: (i, 0)),
        out_shape=jax.ShapeDtypeStruct((_N, _NF), jnp.float32),
    )(h, p0, p1, ut, vt, b0, w1t, b1)


def _coord_edge_body(c0_ref, geo_ref, ct_ref, w1t_ref, b1_ref, w2t_ref, out_ref):
    geo = geo_ref[...]
    ea = geo[:, 0:2]
    t0 = c0_ref[...].astype(jnp.float32) + jnp.dot(ea, ct_ref[...], preferred_element_type=jnp.float32)
    t0 = t0 * jax.nn.sigmoid(t0)
    t1 = jnp.dot(t0, w1t_ref[...], preferred_element_type=jnp.float32) + b1_ref[...]
    t1 = t1 * jax.nn.sigmoid(t1)
    tt = jnp.dot(t1, w2t_ref[...], preferred_element_type=jnp.float32)
    z = jnp.zeros((geo.shape[0], 13), jnp.float32)
    out_ref[...] = jnp.concatenate([geo[:, 2:5] * tt, z], axis=1)


def _tc_coord_edge(c0, geo, ct, w1t, b1, w2t):
    return pl.pallas_call(
        _coord_edge_body,
        grid=(_E // _BE,),
        in_specs=[
            pl.BlockSpec((_BE, _NF), lambda i: (i, 0)),
            pl.BlockSpec((_BE, 16), lambda i: (i, 0)),
            pl.BlockSpec((2, _NF), lambda i: (0, 0)),
            pl.BlockSpec((_NF, _NF), lambda i: (0, 0)),
            pl.BlockSpec((1, _NF), lambda i: (0, 0)),
            pl.BlockSpec((_NF, 1), lambda i: (0, 0)),
        ],
        out_specs=pl.BlockSpec((_BE, 16), lambda i: (i, 0)),
        out_shape=jax.ShapeDtypeStruct((_E, 16), jnp.float32),
    )(c0, geo, ct, w1t, b1, w2t)


def _coord_apply_body(x_ref, q0_ref, q1_ref, out_ref):
    q = (q0_ref[...].astype(jnp.float32)
         + q1_ref[...].astype(jnp.float32)) * (1.0 / _NORM)
    out_ref[...] = x_ref[...] + q[:, 0:3]


def _tc_coord_apply(x, q0, q1):
    return pl.pallas_call(
        _coord_apply_body,
        grid=(_N // _BN,),
        in_specs=[
            pl.BlockSpec((_BN, 3), lambda i: (i, 0)),
            pl.BlockSpec((_BN, 16), lambda i: (i, 0)),
            pl.BlockSpec((_BN, 16), lambda i: (i, 0)),
        ],
        out_specs=pl.BlockSpec((_BN, 3), lambda i: (i, 0)),
        out_shape=jax.ShapeDtypeStruct((_N, 3), jnp.float32),
    )(x, q0, q1)


def _pack_bf16(a):
    return jax.lax.bitcast_convert_type(
        a.reshape(a.shape[0], a.shape[1] // 2, 2), jnp.int32)


def _unpack_bf16(a):
    return jax.lax.bitcast_convert_type(a, jnp.bfloat16).reshape(a.shape[0], -1)


def kernel(h, x, edge_index, edge_attr, params):
    row = edge_index[0]
    col = edge_index[1]
    x128 = jnp.concatenate([x, jnp.zeros((_N, _NF - 3), jnp.float32)], axis=1)
    zeros128 = jnp.zeros((_RPT + 16, _NF), jnp.float32)
    zeros16 = jnp.zeros((_RPT + 16, 16), jnp.float32)

    def ew(i):
        w0 = params[f"gcl{i}_e_W0"]
        return (w0[:, :_NF].T, w0[:, _NF:2 * _NF].T, w0[:, 2 * _NF:].T,
                params[f"gcl{i}_e_b0"][None, :])

    at0, bt0, ct0, ba0 = ew(0)
    at1, bt1, ct1, ba1 = ew(1)
    cw0 = params["c_W0"]
    cat, cbt, cct, cba = (cw0[:, :_NF].T, cw0[:, _NF:2 * _NF].T,
                          cw0[:, 2 * _NF:].T, params["c_b0"][None, :])

    ha0, hb0 = _tc_node_pre(h, at0, bt0, ba0)
    e0, xd = _sc_gather_l0(ha0, hb0, x128, row, col)
    ef0, geo = _tc_edge_l0(
        xd, edge_attr, e0, ct0,
        params["gcl0_e_W1"].T, params["gcl0_e_b1"][None, :],
        params["gcl0_att_W"].T, params["gcl0_att_b"][None, :])
    parts0 = _sc_scatter_add(ef0, row, zeros128, _NF, jnp.float32)
    nw0 = params["gcl0_n_W0"]
    h1, ha1, hb1 = _tc_node_mlp_fused(
        h, parts0[0], parts0[1],
        nw0[:, :_NF].T, nw0[:, _NF:].T, params["gcl0_n_b0"][None, :],
        params["gcl0_n_W1"].T, params["gcl0_n_b1"][None, :],
        at1, bt1, ba1)

    e1 = _sc_gather_combine(ha1, hb1, row, col, _NF, 1, jnp.float32)
    ef1 = _tc_edge_mlp(
        e1, geo, ct1,
        params["gcl1_e_W1"].T, params["gcl1_e_b1"][None, :],
        params["gcl1_att_W"].T, params["gcl1_att_b"][None, :])
    parts1 = _sc_scatter_add(ef1, row, zeros128, _NF, jnp.float32)
    nw1 = params["gcl1_n_W0"]
    h2, ca, cb = _tc_node_mlp_fused(
        h1, parts1[0], parts1[1],
        nw1[:, :_NF].T, nw1[:, _NF:].T, params["gcl1_n_b0"][None, :],
        params["gcl1_n_W1"].T, params["gcl1_n_b1"][None, :],
        cat, cbt, cba)

    c0 = _sc_gather_combine(ca, cb, row, col, _NF, 1, jnp.float32)
    trans = _tc_coord_edge(
        c0, geo, cct,
        params["c_W1"].T,
        params["c_b1"][None, :],
        params["c_W2"].T,
    )
    qparts = _sc_scatter_add(trans, row, zeros16, 16, jnp.float32,
                             tct=False)
    x_new = _tc_coord_apply(x, qparts[0], qparts[1])
    return h2, x_new
